# Initial kernel scaffold; baseline (speedup 1.0000x reference)
import jax
import jax.numpy as jnp
from jax.experimental import pallas as pl


def kernel(state, embeddings, q_values, learning):
    raise NotImplementedError("write your pallas kernel here")


# speedup vs baseline: 6.9220x; 6.9220x over previous
import functools

import jax
import jax.numpy as jnp
from jax import lax
from jax.experimental import pallas as pl
from jax.experimental.pallas import tpu as pltpu
from jax.experimental.pallas import tpu_sc as plsc

N = 100000
D = 128
QN = 1024
KTOP = 32
SEG = 128
NSEG = 784
NPAD = NSEG * SEG
ST = 2048
NST = NPAD // ST
QT = 256
NQT = QN // QT
CW = KTOP * SEG
B = QN * KTOP
NC, NSUB = 2, 16
NW = NC * NSUB
BPW = B // NW
ICH = BPW // 128
HB = BPW // 2
BIG = 1e30


def _dist_body(state_ref, emb_ref, d_ref, segmin_ref, asum_ref):
    qi = pl.program_id(0)
    si = pl.program_id(1)
    x = state_ref[...]
    e = emb_ref[...]
    dot = lax.dot_general(x, e, (((1,), (1,)), ((), ())),
                          preferred_element_type=jnp.float32)
    q2 = jnp.sum(x * x, axis=1, keepdims=True)
    e2 = jnp.sum(e * e, axis=1)[None, :]
    d2 = q2 + e2 - 2.0 * dot
    dist = jnp.sqrt(jnp.maximum(d2, 1e-12))

    @pl.when((qi == 0) & (si == 0))
    def _():
        asum_ref[0, 0] = 0.0

    @pl.when(si < NST - 1)
    def _():
        d_ref[...] = dist
        segmin_ref[0] = jnp.min(dist.reshape(QT, ST // SEG, SEG), axis=2)
        asum_ref[0, 0] += jnp.sum(dist)

    @pl.when(si == NST - 1)
    def _():
        col = si * ST + lax.broadcasted_iota(jnp.int32, (QT, ST), 1)
        pad = col >= N
        dist_m = jnp.where(pad, BIG, dist)
        d_ref[...] = dist_m
        segmin_ref[0] = jnp.min(dist_m.reshape(QT, ST // SEG, SEG), axis=2)
        asum_ref[0, 0] += jnp.sum(jnp.where(pad, 0.0, dist))


def _distances(state, emb_pad):
    return pl.pallas_call(
        _dist_body,
        grid=(NQT, NST),
        in_specs=[
            pl.BlockSpec((QT, D), lambda qi, si: (qi, 0)),
            pl.BlockSpec((ST, D), lambda qi, si: (si, 0)),
        ],
        out_specs=[
            pl.BlockSpec((QT, ST), lambda qi, si: (qi, si)),
            pl.BlockSpec((1, QT, ST // SEG), lambda qi, si: (si, qi, 0)),
            pl.BlockSpec((1, 1), lambda qi, si: (0, 0),
                         memory_space=pltpu.SMEM),
        ],
        out_shape=[
            jax.ShapeDtypeStruct((QN, NPAD), jnp.float32),
            jax.ShapeDtypeStruct((NST, QN, ST // SEG), jnp.float32),
            jax.ShapeDtypeStruct((1, 1), jnp.float32),
        ],
    )(state, emb_pad)


def _select_body(segmin_ref, rowidx_ref, segidx_ref, scr):
    qi = pl.program_id(0)
    scr[...] = segmin_ref[...]
    colio = lax.broadcasted_iota(jnp.int32, (QT, NSEG), 1)
    kio = lax.broadcasted_iota(jnp.int32, (QT, KTOP), 1)

    def step(k, acc):
        m = scr[...]
        mn = jnp.min(m, axis=1, keepdims=True)
        ii = jnp.min(jnp.where(m == mn, colio, NSEG), axis=1, keepdims=True)
        acc = jnp.where(kio == k, ii, acc)
        scr[...] = jnp.where(colio == ii, BIG, m)
        return acc

    acc = lax.fori_loop(0, KTOP, step, jnp.zeros((QT, KTOP), jnp.int32))
    segidx_ref[...] = acc
    qrow = qi * QT + lax.broadcasted_iota(jnp.int32, (QT, KTOP), 0)
    rowidx_ref[...] = acc + qrow * NSEG


def _select(segmin):
    return pl.pallas_call(
        _select_body,
        grid=(NQT,),
        in_specs=[pl.BlockSpec((QT, NSEG), lambda qi: (qi, 0))],
        out_specs=[
            pl.BlockSpec((QT, KTOP), lambda qi: (qi, 0)),
            pl.BlockSpec((QT, KTOP), lambda qi: (qi, 0)),
        ],
        out_shape=[
            jax.ShapeDtypeStruct((QN, KTOP), jnp.int32),
            jax.ShapeDtypeStruct((QN, KTOP), jnp.int32),
        ],
        scratch_shapes=[pltpu.VMEM((QT, NSEG), jnp.float32)],
    )(segmin)


_SC_MESH = plsc.VectorSubcoreMesh(core_axis_name="c", subcore_axis_name="s")


@functools.partial(
    pl.kernel,
    mesh=_SC_MESH,
    out_type=[
        jax.ShapeDtypeStruct((B, SEG), jnp.float32),
        jax.ShapeDtypeStruct((B, SEG), jnp.float32),
    ],
    scratch_types=[
        pltpu.VMEM((ICH, 128), jnp.int32),
        pltpu.VMEM((HB, SEG), jnp.float32),
        pltpu.SemaphoreType.DMA,
    ],
)
def _sc_gather(drows_hbm, qvseg_hbm, rowidx_hbm, segidx_hbm,
               dcand_hbm, vcand_hbm, idx_v, rows_v, sem):
    wid = lax.axis_index("s") * NC + lax.axis_index("c")
    base = wid * BPW
    for table, idx_hbm, out_hbm in (
        (drows_hbm, rowidx_hbm, dcand_hbm),
        (qvseg_hbm, segidx_hbm, vcand_hbm),
    ):
        pltpu.sync_copy(idx_hbm.at[wid], idx_v)
        for h in range(2):
            cps = [
                pltpu.async_copy(table.at[idx_v.at[h * (ICH // 2) + j]],
                                 rows_v.at[pl.ds(j * 128, 128)], sem)
                for j in range(ICH // 2)
            ]
            for c in cps:
                c.wait()
            pltpu.sync_copy(rows_v, out_hbm.at[pl.ds(base + h * HB, HB)])


def _final_body(dc_ref, vc_ref, asum_ref, out_ref, dscr):
    qi = pl.program_id(0)

    @pl.when(qi == 0)
    def _():
        out_ref[0, 0] = 0.0

    dscr[...] = dc_ref[...]
    colio = lax.broadcasted_iota(jnp.int32, (QT, CW), 1)

    def step(k, acc):
        dmat = dscr[...]
        vmat = vc_ref[...]
        mn = jnp.min(dmat, axis=1, keepdims=True)
        ii = jnp.min(jnp.where(dmat == mn, colio, CW), axis=1, keepdims=True)
        sel = colio == ii
        val = jnp.sum(jnp.where(sel, vmat, 0.0), axis=1, keepdims=True)
        dscr[...] = jnp.where(sel, BIG, dmat)
        return acc + (1.0 - mn) * val

    acc = lax.fori_loop(0, KTOP, step, jnp.zeros((QT, 1), jnp.float32))
    out_ref[0, 0] += jnp.sum(acc)

    @pl.when(qi == NQT - 1)
    def _():
        out_ref[0, 0] = out_ref[0, 0] / asum_ref[0, 0]


def _combine(dcand, vcand, asum):
    return pl.pallas_call(
        _final_body,
        grid=(NQT,),
        in_specs=[
            pl.BlockSpec((QT, CW), lambda qi: (qi, 0)),
            pl.BlockSpec((QT, CW), lambda qi: (qi, 0)),
            pl.BlockSpec((1, 1), lambda qi: (0, 0),
                         memory_space=pltpu.SMEM),
        ],
        out_specs=pl.BlockSpec((1, 1), lambda qi: (0, 0),
                               memory_space=pltpu.SMEM),
        out_shape=jax.ShapeDtypeStruct((1, 1), jnp.float32),
        scratch_shapes=[pltpu.VMEM((QT, CW), jnp.float32)],
    )(dcand, vcand, asum)


def kernel(state, embeddings, q_values, learning):
    qv_seg = jnp.pad(q_values, (0, NPAD - N)).reshape(NSEG, SEG)
    d, segmin3, asum = _distances(state, embeddings)
    segmin = segmin3.transpose(1, 0, 2).reshape(QN, NSEG)
    rowidx, segidx = _select(segmin)
    dcand, vcand = _sc_gather(
        d.reshape(QN * NSEG, SEG), qv_seg,
        rowidx.reshape(NW, ICH, 128), segidx.reshape(NW, ICH, 128))
    out = _combine(dcand.reshape(QN, CW), vcand.reshape(QN, CW), asum)
    return out[0, 0] + jnp.asarray(learning, jnp.float32) * 0.0
